# baseline (speedup 1.0000x reference)
import jax
import jax.numpy as jnp
import numpy as np
from jax import lax
from jax.experimental import pallas as pl
from jax.experimental.pallas import tpu as pltpu
from jax.experimental.pallas import tpu_sc as plsc

N = 100000
E = 1600000
R = 8
NC = 2
NS = 16
NP = 4
K = N * R // (NC * NP)
DUM = 128
ET = E // NS
EB = 2000
NB = ET // EB
NG = EB // 16
NGP = 128
NCH = NGP * 16 // 128
WB = 200
D = 5
NWB = K // WB
BT = 1000
_z = np.int32(0)


def _make_sc_agg(with_cnt):
  mesh = plsc.VectorSubcoreMesh(core_axis_name="c", subcore_axis_name="s",
                                num_cores=NC, num_subcores=NS)
  out_type = [jax.ShapeDtypeStruct((N * R, 16), jnp.float32)]
  if with_cnt:
    out_type.append(jax.ShapeDtypeStruct((N * R,), jnp.float32))
  scratch = [
      pltpu.VMEM((2 * NGP * 16,), jnp.int32),
      pltpu.VMEM((2 * NGP * 16,), jnp.int32),
      pltpu.VMEM((NCH, 128), jnp.int32),
      pltpu.VMEM((D, 128, 16), jnp.float32),
      pltpu.VMEM((WB, 16), jnp.float32),
      pltpu.VMEM_SHARED((K + DUM, 16), jnp.float32),
  ]
  if with_cnt:
    scratch += [
        pltpu.VMEM((128,), jnp.float32),
        pltpu.VMEM((WB,), jnp.float32),
        pltpu.VMEM_SHARED((K + DUM,), jnp.float32),
    ]
  for _ in range(D):
    scratch.append(pltpu.SemaphoreType.DMA)
  scratch.append(pltpu.SemaphoreType.DMA)

  def body(*args):
    if with_cnt:
      (skey_h, gidx_h, tab_h, z2_h, z1_h, s_out, c_out,
       kbuf, gbuf, sidx, rows, zb2, stab, ones, zb1, ctab,
       *sems) = args
    else:
      (skey_h, gidx_h, tab_h, z2_h, s_out,
       kbuf, gbuf, sidx, rows, zb2, stab, *sems) = args
    ring = sems[:D]
    semi = sems[D]
    c = lax.axis_index("c")
    s = lax.axis_index("s")
    iota = lax.iota(jnp.int32, 16)
    SLOT = NGP * 16
    pltpu.sync_copy(z2_h, zb2)
    if with_cnt:
      pltpu.sync_copy(z1_h, zb1)
      for i in range(8):
        ones[pl.ds(i * 16, 16)] = jnp.ones((16,), jnp.float32)
    for s2 in range(2):
      for g in range(NG, NGP):
        gbuf[pl.ds(s2 * SLOT + g * 16, 16)] = iota + (g * 16)
    for g in range(NG, NGP):
      sidx[jnp.int32(g // 8), pl.ds((g % 8) * 16, 16)] = (
          (K + (g % 8) * 16) + iota)

    def stage(bi, slot):
      off = s * ET + bi * EB
      pltpu.async_copy(skey_h.at[pl.ds(off, EB)],
                       kbuf.at[pl.ds(slot * SLOT, EB)], semi)
      pltpu.async_copy(gidx_h.at[pl.ds(off, EB)],
                       gbuf.at[pl.ds(slot * SLOT, EB)], semi)

    def drain_stage():
      pltpu.make_async_copy(skey_h.at[pl.ds(0, EB)],
                            kbuf.at[pl.ds(0, EB)], semi).wait()
      pltpu.make_async_copy(gidx_h.at[pl.ds(0, EB)],
                            gbuf.at[pl.ds(0, EB)], semi).wait()

    def one_pass(p, _):
      base = (c * NP + p) * K

      def zero_body(i, _):
        j = s + i * NS

        @pl.when(j < NWB)
        def _():
          pltpu.sync_copy(zb2, stab.at[pl.ds(j * WB, WB)])
          if with_cnt:
            pltpu.sync_copy(zb1, ctab.at[pl.ds(j * WB, WB)])
        return jnp.int32(0)

      lax.fori_loop(jnp.int32(0), jnp.int32((NWB + NS - 1) // NS), zero_body, jnp.int32(0))
      plsc.subcore_barrier()
      stage(jnp.int32(0), 0)

      def do_block(b, slot):
        drain_stage()

        def prep(j):
          for g in range(j * 8, min((j + 1) * 8, NG)):
            dummy = (K + (g % 8) * 16) + iota
            o = slot * SLOT + g * 16
            loc = kbuf[pl.ds(o, 16)] - base
            ok = (loc >= 0) & (loc < K)
            sidx[jnp.int32(g // 8), pl.ds((g % 8) * 16, 16)] = (
                jnp.where(ok, loc, dummy))
            gbuf[pl.ds(o, 16)] = jnp.where(ok, gbuf[pl.ds(o, 16)], _z)

        def fire(j):
          prep(j)
          return pltpu.async_copy(
              tab_h.at[gbuf.at[pl.ds(slot * SLOT + j * 128, 128)]],
              rows.at[np.int32(j % D)], ring[j % D])

        def scat(j, cp):
          cp.wait()
          pltpu.sync_copy(rows.at[np.int32(j % D)],
                          stab.at[sidx.at[jnp.int32(j)]], add=True)
          if with_cnt:
            pltpu.sync_copy(ones, ctab.at[sidx.at[jnp.int32(j)]], add=True)

        cps = [fire(j) for j in range(D)]

        @pl.when(b + 1 < NB)
        def _():
          stage(b + 1, 1 - slot)
        for j in range(D, NCH):
          scat(j - D, cps[j - D])
          cps.append(fire(j))
        for j in range(NCH - D, NCH):
          scat(j, cps[j])

      def blk2(i, _):
        do_block(i * 2, 0)
        do_block(i * 2 + 1, 1)
        return jnp.int32(0)

      lax.fori_loop(jnp.int32(0), jnp.int32(NB // 2), blk2, jnp.int32(0))
      plsc.subcore_barrier()

      def wb_body(i, _):
        j = s + i * NS

        @pl.when(j < NWB)
        def _():
          pltpu.sync_copy(stab.at[pl.ds(j * WB, WB)],
                          s_out.at[pl.ds(base + j * WB, WB)])
          if with_cnt:
            pltpu.sync_copy(ctab.at[pl.ds(j * WB, WB)],
                            c_out.at[pl.ds(base + j * WB, WB)])
        return jnp.int32(0)

      lax.fori_loop(jnp.int32(0), jnp.int32((NWB + NS - 1) // NS), wb_body, jnp.int32(0))
      plsc.subcore_barrier()
      return jnp.int32(0)

    lax.fori_loop(jnp.int32(0), jnp.int32(NP), one_pass, jnp.int32(0))

  return pl.kernel(
      body, out_type=out_type, mesh=mesh, scratch_types=scratch,
      compiler_params=pltpu.CompilerParams(use_tc_tiling_on_sc=False))


_sc_l1 = _make_sc_agg(True)
_sc_l2 = _make_sc_agg(False)


def _tc1_body(s_ref, c_ref, x_ref, w1_ref, r1_ref, b1_ref, w2_ref,
              h_ref, h2_ref):
  inv = 1.0 / jnp.maximum(c_ref[...], 1.0)
  acc = jnp.dot(x_ref[...], r1_ref[...],
                preferred_element_type=jnp.float32) + b1_ref[...]
  sv = s_ref[...]
  for r in range(R):
    acc = acc + jnp.dot(sv[:, r * 16:(r + 1) * 16] * inv[:, r:r + 1],
                        w1_ref[r], preferred_element_type=jnp.float32)
  h = jnp.maximum(acc, 0.0)
  h_ref[...] = h
  for r in range(R):
    h2_ref[:, r * 16:(r + 1) * 16] = jnp.dot(
        h, w2_ref[r], preferred_element_type=jnp.float32)


_tc1 = pl.pallas_call(
    _tc1_body,
    grid=(N // BT,),
    in_specs=[
        pl.BlockSpec((BT, 128), lambda i: (i, _z)),
        pl.BlockSpec((BT, R), lambda i: (i, _z)),
        pl.BlockSpec((BT, 16), lambda i: (i, _z)),
        pl.BlockSpec((R, 16, 32), lambda i: (_z, _z, _z)),
        pl.BlockSpec((16, 32), lambda i: (_z, _z)),
        pl.BlockSpec((1, 32), lambda i: (_z, _z)),
        pl.BlockSpec((R, 32, 16), lambda i: (_z, _z, _z)),
    ],
    out_specs=[
        pl.BlockSpec((BT, 32), lambda i: (i, _z)),
        pl.BlockSpec((BT, 128), lambda i: (i, _z)),
    ],
    out_shape=[
        jax.ShapeDtypeStruct((N, 32), jnp.float32),
        jax.ShapeDtypeStruct((N, 128), jnp.float32),
    ],
)


def _tc2_body(t_ref, c_ref, h_ref, r2_ref, b2_ref, o_ref):
  inv = 1.0 / jnp.maximum(c_ref[...], 1.0)
  acc = jnp.dot(h_ref[...], r2_ref[...],
                preferred_element_type=jnp.float32) + b2_ref[...]
  tv = t_ref[...]
  for r in range(R):
    acc = acc + tv[:, r * 16:(r + 1) * 16] * inv[:, r:r + 1]
  o_ref[...] = acc


_tc2 = pl.pallas_call(
    _tc2_body,
    grid=(N // BT,),
    in_specs=[
        pl.BlockSpec((BT, 128), lambda i: (i, _z)),
        pl.BlockSpec((BT, R), lambda i: (i, _z)),
        pl.BlockSpec((BT, 32), lambda i: (i, _z)),
        pl.BlockSpec((32, 16), lambda i: (_z, _z)),
        pl.BlockSpec((1, 16), lambda i: (_z, _z)),
    ],
    out_specs=pl.BlockSpec((BT, 16), lambda i: (i, _z)),
    out_shape=jax.ShapeDtypeStruct((N, 16), jnp.float32),
)


def kernel(x, edge_index, edge_type, w1, root1, b1, w2, root2, b2):
  src = edge_index[0].astype(jnp.int32)
  dst = edge_index[1].astype(jnp.int32)
  rel = edge_type.astype(jnp.int32)
  skey = dst * R + rel
  g2 = src * R + rel
  z2 = jnp.zeros((WB, 16), jnp.float32)
  z1 = jnp.zeros((WB,), jnp.float32)
  S1, cnt = _sc_l1(skey, src, x, z2, z1)
  cnt2 = cnt.reshape(N, R)
  h, h2 = _tc1(S1.reshape(N, 128), cnt2, x, w1, root1, b1.reshape(1, 32), w2)
  T2 = _sc_l2(skey, g2, h2.reshape(N * R, 16), z2)
  if isinstance(T2, (list, tuple)):
    T2 = T2[0]
  out = _tc2(T2.reshape(N, 128), cnt2, h, root2, b2.reshape(1, 16))
  return out


# speedup vs baseline: 42.9806x; 42.9806x over previous
import jax
import jax.numpy as jnp
import numpy as np
from jax import lax
from jax.experimental import pallas as pl
from jax.experimental.pallas import tpu as pltpu
from jax.experimental.pallas import tpu_sc as plsc

N = 100000
E = 1600000
R = 8
NC = 2
NS = 16
NP = 2
K = N * R // (NC * NP)
DUM = 128
ET = E // NS
EB = 2000
NB = ET // EB
NG = EB // 16
NGP = 128
NCH = NGP * 16 // 128
WB = 200
D = 5
NWB = K // WB
BT = 1000
_z = np.int32(0)


def _make_sc_agg(with_cnt):
  mesh = plsc.VectorSubcoreMesh(core_axis_name="c", subcore_axis_name="s",
                                num_cores=NC, num_subcores=NS)
  out_type = [jax.ShapeDtypeStruct((N * R, 16), jnp.bfloat16)]
  if with_cnt:
    out_type.append(jax.ShapeDtypeStruct((N * R,), jnp.float32))
  scratch = [
      pltpu.VMEM((2 * NGP * 16,), jnp.int32),
      pltpu.VMEM((2 * NGP * 16,), jnp.int32),
      pltpu.VMEM((NCH, 128), jnp.int32),
      pltpu.VMEM((D, 128, 16), jnp.bfloat16),
      pltpu.VMEM((WB, 16), jnp.bfloat16),
      pltpu.VMEM_SHARED((K + DUM, 16), jnp.bfloat16),
  ]
  if with_cnt:
    scratch += [
        pltpu.VMEM((128,), jnp.float32),
        pltpu.VMEM((WB,), jnp.float32),
        pltpu.VMEM_SHARED((K + DUM,), jnp.float32),
    ]
  for _ in range(D):
    scratch.append(pltpu.SemaphoreType.DMA)
  scratch.append(pltpu.SemaphoreType.DMA)

  def body(*args):
    if with_cnt:
      (skey_h, gidx_h, tab_h, z2_h, z1_h, s_out, c_out,
       kbuf, gbuf, sidx, rows, zb2, stab, ones, zb1, ctab,
       *sems) = args
    else:
      (skey_h, gidx_h, tab_h, z2_h, s_out,
       kbuf, gbuf, sidx, rows, zb2, stab, *sems) = args
    ring = sems[:D]
    semi = sems[D]
    c = lax.axis_index("c")
    s = lax.axis_index("s")
    iota = lax.iota(jnp.int32, 16)
    SLOT = NGP * 16
    pltpu.sync_copy(z2_h, zb2)
    if with_cnt:
      pltpu.sync_copy(z1_h, zb1)
      for i in range(8):
        ones[pl.ds(i * 16, 16)] = jnp.ones((16,), jnp.float32)
    for s2 in range(2):
      for g in range(NG, NGP):
        gbuf[pl.ds(s2 * SLOT + g * 16, 16)] = iota + (g * 16)
    for g in range(NG, NGP):
      sidx[jnp.int32(g // 8), pl.ds((g % 8) * 16, 16)] = (
          (K + (g % 8) * 16) + iota)

    def stage(bi, slot):
      off = s * ET + bi * EB
      pltpu.async_copy(skey_h.at[pl.ds(off, EB)],
                       kbuf.at[pl.ds(slot * SLOT, EB)], semi)
      pltpu.async_copy(gidx_h.at[pl.ds(off, EB)],
                       gbuf.at[pl.ds(slot * SLOT, EB)], semi)

    def drain_stage():
      pltpu.make_async_copy(skey_h.at[pl.ds(0, EB)],
                            kbuf.at[pl.ds(0, EB)], semi).wait()
      pltpu.make_async_copy(gidx_h.at[pl.ds(0, EB)],
                            gbuf.at[pl.ds(0, EB)], semi).wait()

    def one_pass(p, _):
      base = (c * NP + p) * K

      def zero_body(i, _):
        j = s + i * NS

        @pl.when(j < NWB)
        def _():
          pltpu.sync_copy(zb2, stab.at[pl.ds(j * WB, WB)])
          if with_cnt:
            pltpu.sync_copy(zb1, ctab.at[pl.ds(j * WB, WB)])
        return jnp.int32(0)

      lax.fori_loop(jnp.int32(0), jnp.int32((NWB + NS - 1) // NS), zero_body, jnp.int32(0))
      plsc.subcore_barrier()
      stage(jnp.int32(0), 0)

      def do_block(b, slot):
        drain_stage()

        def fire(j):
          return pltpu.async_copy(
              tab_h.at[gbuf.at[pl.ds(slot * SLOT + j * 128, 128)]],
              rows.at[np.int32(j % D)], ring[j % D])

        def scat(j, cp):
          cp.wait()
          pltpu.sync_copy(rows.at[np.int32(j % D)],
                          stab.at[sidx.at[jnp.int32(j)]], add=True)
          if with_cnt:
            pltpu.sync_copy(ones, ctab.at[sidx.at[jnp.int32(j)]], add=True)

        cps = [fire(j) for j in range(D)]

        @pl.when(b + 1 < NB)
        def _():
          stage(b + 1, 1 - slot)
        for g in range(NG):
          dummy = (K + (g % 8) * 16) + iota
          loc = kbuf[pl.ds(slot * SLOT + g * 16, 16)] - base
          ok = (loc >= 0) & (loc < K)
          sidx[jnp.int32(g // 8), pl.ds((g % 8) * 16, 16)] = (
              jnp.where(ok, loc, dummy))
        for j in range(D, NCH):
          scat(j - D, cps[j - D])
          cps.append(fire(j))
        for j in range(NCH - D, NCH):
          scat(j, cps[j])

      def blk2(i, _):
        do_block(i * 2, 0)
        do_block(i * 2 + 1, 1)
        return jnp.int32(0)

      lax.fori_loop(jnp.int32(0), jnp.int32(NB // 2), blk2, jnp.int32(0))
      plsc.subcore_barrier()

      def wb_body(i, _):
        j = s + i * NS

        @pl.when(j < NWB)
        def _():
          pltpu.sync_copy(stab.at[pl.ds(j * WB, WB)],
                          s_out.at[pl.ds(base + j * WB, WB)])
          if with_cnt:
            pltpu.sync_copy(ctab.at[pl.ds(j * WB, WB)],
                            c_out.at[pl.ds(base + j * WB, WB)])
        return jnp.int32(0)

      lax.fori_loop(jnp.int32(0), jnp.int32((NWB + NS - 1) // NS), wb_body, jnp.int32(0))
      plsc.subcore_barrier()
      return jnp.int32(0)

    lax.fori_loop(jnp.int32(0), jnp.int32(NP), one_pass, jnp.int32(0))

  return pl.kernel(
      body, out_type=out_type, mesh=mesh, scratch_types=scratch,
      compiler_params=pltpu.CompilerParams(use_tc_tiling_on_sc=False))


_sc_l1 = _make_sc_agg(True)
_sc_l2 = _make_sc_agg(False)


def _tc1_body(s_ref, c_ref, x_ref, w1_ref, r1_ref, b1_ref, w2_ref,
              h_ref, h2_ref):
  inv = 1.0 / jnp.maximum(c_ref[...], 1.0)
  acc = jnp.dot(x_ref[...], r1_ref[...],
                preferred_element_type=jnp.float32) + b1_ref[...]
  sv = s_ref[...].astype(jnp.float32)
  for r in range(R):
    acc = acc + jnp.dot(sv[:, r * 16:(r + 1) * 16] * inv[:, r:r + 1],
                        w1_ref[r], preferred_element_type=jnp.float32)
  h = jnp.maximum(acc, 0.0)
  h_ref[...] = h
  for r in range(R):
    h2_ref[:, r * 16:(r + 1) * 16] = jnp.dot(
        h, w2_ref[r], preferred_element_type=jnp.float32).astype(jnp.bfloat16)


_tc1 = pl.pallas_call(
    _tc1_body,
    grid=(N // BT,),
    in_specs=[
        pl.BlockSpec((BT, 128), lambda i: (i, _z)),
        pl.BlockSpec((BT, R), lambda i: (i, _z)),
        pl.BlockSpec((BT, 16), lambda i: (i, _z)),
        pl.BlockSpec((R, 16, 32), lambda i: (_z, _z, _z)),
        pl.BlockSpec((16, 32), lambda i: (_z, _z)),
        pl.BlockSpec((1, 32), lambda i: (_z, _z)),
        pl.BlockSpec((R, 32, 16), lambda i: (_z, _z, _z)),
    ],
    out_specs=[
        pl.BlockSpec((BT, 32), lambda i: (i, _z)),
        pl.BlockSpec((BT, 128), lambda i: (i, _z)),
    ],
    out_shape=[
        jax.ShapeDtypeStruct((N, 32), jnp.float32),
        jax.ShapeDtypeStruct((N, 128), jnp.bfloat16),
    ],
)


def _tc2_body(t_ref, c_ref, h_ref, r2_ref, b2_ref, o_ref):
  inv = 1.0 / jnp.maximum(c_ref[...], 1.0)
  acc = jnp.dot(h_ref[...], r2_ref[...],
                preferred_element_type=jnp.float32) + b2_ref[...]
  tv = t_ref[...].astype(jnp.float32)
  for r in range(R):
    acc = acc + tv[:, r * 16:(r + 1) * 16] * inv[:, r:r + 1]
  o_ref[...] = acc


_tc2 = pl.pallas_call(
    _tc2_body,
    grid=(N // BT,),
    in_specs=[
        pl.BlockSpec((BT, 128), lambda i: (i, _z)),
        pl.BlockSpec((BT, R), lambda i: (i, _z)),
        pl.BlockSpec((BT, 32), lambda i: (i, _z)),
        pl.BlockSpec((32, 16), lambda i: (_z, _z)),
        pl.BlockSpec((1, 16), lambda i: (_z, _z)),
    ],
    out_specs=pl.BlockSpec((BT, 16), lambda i: (i, _z)),
    out_shape=jax.ShapeDtypeStruct((N, 16), jnp.float32),
)


def kernel(x, edge_index, edge_type, w1, root1, b1, w2, root2, b2):
  src = edge_index[0].astype(jnp.int32)
  dst = edge_index[1].astype(jnp.int32)
  rel = edge_type.astype(jnp.int32)
  skey = dst * R + rel
  g2 = src * R + rel
  z2 = jnp.zeros((WB, 16), jnp.bfloat16)
  z1 = jnp.zeros((WB,), jnp.float32)
  S1, cnt = _sc_l1(skey, src, x.astype(jnp.bfloat16), z2, z1)
  cnt2 = cnt.reshape(N, R)
  h, h2 = _tc1(S1.reshape(N, 128), cnt2, x, w1, root1, b1.reshape(1, 32), w2)
  T2 = _sc_l2(skey, g2, h2.reshape(N * R, 16), z2)
  if isinstance(T2, (list, tuple)):
    T2 = T2[0]
  out = _tc2(T2.reshape(N, 128), cnt2, h, root2, b2.reshape(1, 16))
  return out


# speedup vs baseline: 46.7540x; 1.0878x over previous
import jax
import jax.numpy as jnp
import numpy as np
from jax import lax
from jax.experimental import pallas as pl
from jax.experimental.pallas import tpu as pltpu
from jax.experimental.pallas import tpu_sc as plsc

N = 100000
E = 1600000
R = 8
NC = 2
NS = 16
NP = 2
K = N * R // (NC * NP)
DUM = 128
ET = E // NS
EB = 2000
NB = ET // EB
NG = EB // 16
NGP = 128
NCH = NGP * 16 // 128
WB = 200
D = 5
NWB = K // WB
BT = 1000
_z = np.int32(0)


def _make_sc_agg(with_cnt):
  mesh = plsc.VectorSubcoreMesh(core_axis_name="c", subcore_axis_name="s",
                                num_cores=NC, num_subcores=NS)
  out_type = [jax.ShapeDtypeStruct((N * R, 16), jnp.bfloat16)]
  if with_cnt:
    out_type.append(jax.ShapeDtypeStruct((N * R,), jnp.float32))
  scratch = [
      pltpu.VMEM((2 * NGP * 16,), jnp.int32),
      pltpu.VMEM((2 * NGP * 16,), jnp.int32),
      pltpu.VMEM((NCH, 128), jnp.int32),
      pltpu.VMEM((D, 128, 16), jnp.bfloat16),
      pltpu.VMEM((WB, 16), jnp.bfloat16),
      pltpu.VMEM_SHARED((K + DUM, 16), jnp.bfloat16),
  ]
  if with_cnt:
    scratch += [
        pltpu.VMEM((128,), jnp.float32),
        pltpu.VMEM((WB,), jnp.float32),
        pltpu.VMEM_SHARED((K + DUM,), jnp.float32),
    ]
  for _ in range(D):
    scratch.append(pltpu.SemaphoreType.DMA)
  scratch.append(pltpu.SemaphoreType.DMA)

  def body(*args):
    if with_cnt:
      (skey_h, gidx_h, tab_h, z2_h, z1_h, s_out, c_out,
       kbuf, gbuf, sidx, rows, zb2, stab, ones, zb1, ctab,
       *sems) = args
    else:
      (skey_h, gidx_h, tab_h, z2_h, s_out,
       kbuf, gbuf, sidx, rows, zb2, stab, *sems) = args
    ring = sems[:D]
    semi = sems[D]
    c = lax.axis_index("c")
    s = lax.axis_index("s")
    iota = lax.iota(jnp.int32, 16)
    SLOT = NGP * 16
    pltpu.sync_copy(z2_h, zb2)
    if with_cnt:
      pltpu.sync_copy(z1_h, zb1)
      for i in range(8):
        ones[pl.ds(i * 16, 16)] = jnp.ones((16,), jnp.float32)
    for s2 in range(2):
      for g in range(NG, NGP):
        gbuf[pl.ds(s2 * SLOT + g * 16, 16)] = iota + (g * 16)
    for g in range(NG, NGP):
      sidx[jnp.int32(g // 8), pl.ds((g % 8) * 16, 16)] = (
          (K + (g % 8) * 16) + iota)

    def stage(bi, slot):
      off = s * ET + bi * EB
      pltpu.async_copy(skey_h.at[pl.ds(off, EB)],
                       kbuf.at[pl.ds(slot * SLOT, EB)], semi)
      pltpu.async_copy(gidx_h.at[pl.ds(off, EB)],
                       gbuf.at[pl.ds(slot * SLOT, EB)], semi)

    def drain_stage():
      pltpu.make_async_copy(skey_h.at[pl.ds(0, EB)],
                            kbuf.at[pl.ds(0, EB)], semi).wait()
      pltpu.make_async_copy(gidx_h.at[pl.ds(0, EB)],
                            gbuf.at[pl.ds(0, EB)], semi).wait()

    NIT = (NWB + NS - 1) // NS

    def one_pass(p, _):
      base = (c * NP + p) * K
      stage(jnp.int32(0), 0)

      for i in range(NIT):
        j = s + i * NS

        @pl.when(j < NWB)
        def _(i=i, j=j):
          if i >= D:
            pltpu.make_async_copy(zb2, stab.at[pl.ds(0, WB)],
                                  ring[i % D]).wait()
            if with_cnt:
              pltpu.make_async_copy(zb1, ctab.at[pl.ds(0, WB)],
                                    ring[i % D]).wait()
          pltpu.async_copy(zb2, stab.at[pl.ds(j * WB, WB)], ring[i % D])
          if with_cnt:
            pltpu.async_copy(zb1, ctab.at[pl.ds(j * WB, WB)], ring[i % D])
      for i in range(NIT - D, NIT):
        pltpu.make_async_copy(zb2, stab.at[pl.ds(0, WB)],
                              ring[i % D]).wait()
        if with_cnt:
          pltpu.make_async_copy(zb1, ctab.at[pl.ds(0, WB)],
                                ring[i % D]).wait()
      plsc.subcore_barrier()

      def do_block(b, slot):
        drain_stage()

        def fire(j):
          return pltpu.async_copy(
              tab_h.at[gbuf.at[pl.ds(slot * SLOT + j * 128, 128)]],
              rows.at[np.int32(j % D)], ring[j % D])

        def scat(j, cp):
          cp.wait()
          pltpu.sync_copy(rows.at[np.int32(j % D)],
                          stab.at[sidx.at[jnp.int32(j)]], add=True)
          if with_cnt:
            pltpu.sync_copy(ones, ctab.at[sidx.at[jnp.int32(j)]], add=True)

        cps = [fire(j) for j in range(D)]

        @pl.when(b + 1 < NB)
        def _():
          stage(b + 1, 1 - slot)
        for g in range(NG):
          dummy = (K + (g % 8) * 16) + iota
          loc = kbuf[pl.ds(slot * SLOT + g * 16, 16)] - base
          ok = (loc >= 0) & (loc < K)
          sidx[jnp.int32(g // 8), pl.ds((g % 8) * 16, 16)] = (
              jnp.where(ok, loc, dummy))
        for j in range(D, NCH):
          scat(j - D, cps[j - D])
          cps.append(fire(j))
        for j in range(NCH - D, NCH):
          scat(j, cps[j])

      def blk2(i, _):
        do_block(i * 2, 0)
        do_block(i * 2 + 1, 1)
        return jnp.int32(0)

      lax.fori_loop(jnp.int32(0), jnp.int32(NB // 2), blk2, jnp.int32(0))
      plsc.subcore_barrier()

      for i in range(NIT):
        j = s + i * NS

        @pl.when(j < NWB)
        def _(i=i, j=j):
          if i >= D:
            pltpu.make_async_copy(stab.at[pl.ds(0, WB)],
                                  s_out.at[pl.ds(0, WB)], ring[i % D]).wait()
            if with_cnt:
              pltpu.make_async_copy(ctab.at[pl.ds(0, WB)],
                                    c_out.at[pl.ds(0, WB)], ring[i % D]).wait()
          pltpu.async_copy(stab.at[pl.ds(j * WB, WB)],
                           s_out.at[pl.ds(base + j * WB, WB)], ring[i % D])
          if with_cnt:
            pltpu.async_copy(ctab.at[pl.ds(j * WB, WB)],
                             c_out.at[pl.ds(base + j * WB, WB)], ring[i % D])
      for i in range(NIT - D, NIT):
        pltpu.make_async_copy(stab.at[pl.ds(0, WB)],
                              s_out.at[pl.ds(0, WB)], ring[i % D]).wait()
        if with_cnt:
          pltpu.make_async_copy(ctab.at[pl.ds(0, WB)],
                                c_out.at[pl.ds(0, WB)], ring[i % D]).wait()
      plsc.subcore_barrier()
      return jnp.int32(0)

    lax.fori_loop(jnp.int32(0), jnp.int32(NP), one_pass, jnp.int32(0))

  return pl.kernel(
      body, out_type=out_type, mesh=mesh, scratch_types=scratch,
      compiler_params=pltpu.CompilerParams(use_tc_tiling_on_sc=False))


_sc_l1 = _make_sc_agg(True)
_sc_l2 = _make_sc_agg(False)


def _tc1_body(s_ref, c_ref, x_ref, w1_ref, r1_ref, b1_ref, w2_ref,
              h_ref, h2_ref):
  inv = 1.0 / jnp.maximum(c_ref[...], 1.0)
  acc = jnp.dot(x_ref[...], r1_ref[...],
                preferred_element_type=jnp.float32) + b1_ref[...]
  sv = s_ref[...].astype(jnp.float32)
  for r in range(R):
    acc = acc + jnp.dot(sv[:, r * 16:(r + 1) * 16] * inv[:, r:r + 1],
                        w1_ref[r], preferred_element_type=jnp.float32)
  h = jnp.maximum(acc, 0.0)
  h_ref[...] = h
  for r in range(R):
    h2_ref[:, r * 16:(r + 1) * 16] = jnp.dot(
        h, w2_ref[r], preferred_element_type=jnp.float32).astype(jnp.bfloat16)


_tc1 = pl.pallas_call(
    _tc1_body,
    grid=(N // BT,),
    in_specs=[
        pl.BlockSpec((BT, 128), lambda i: (i, _z)),
        pl.BlockSpec((BT, R), lambda i: (i, _z)),
        pl.BlockSpec((BT, 16), lambda i: (i, _z)),
        pl.BlockSpec((R, 16, 32), lambda i: (_z, _z, _z)),
        pl.BlockSpec((16, 32), lambda i: (_z, _z)),
        pl.BlockSpec((1, 32), lambda i: (_z, _z)),
        pl.BlockSpec((R, 32, 16), lambda i: (_z, _z, _z)),
    ],
    out_specs=[
        pl.BlockSpec((BT, 32), lambda i: (i, _z)),
        pl.BlockSpec((BT, 128), lambda i: (i, _z)),
    ],
    out_shape=[
        jax.ShapeDtypeStruct((N, 32), jnp.float32),
        jax.ShapeDtypeStruct((N, 128), jnp.bfloat16),
    ],
)


def _tc2_body(t_ref, c_ref, h_ref, r2_ref, b2_ref, o_ref):
  inv = 1.0 / jnp.maximum(c_ref[...], 1.0)
  acc = jnp.dot(h_ref[...], r2_ref[...],
                preferred_element_type=jnp.float32) + b2_ref[...]
  tv = t_ref[...].astype(jnp.float32)
  for r in range(R):
    acc = acc + tv[:, r * 16:(r + 1) * 16] * inv[:, r:r + 1]
  o_ref[...] = acc


_tc2 = pl.pallas_call(
    _tc2_body,
    grid=(N // BT,),
    in_specs=[
        pl.BlockSpec((BT, 128), lambda i: (i, _z)),
        pl.BlockSpec((BT, R), lambda i: (i, _z)),
        pl.BlockSpec((BT, 32), lambda i: (i, _z)),
        pl.BlockSpec((32, 16), lambda i: (_z, _z)),
        pl.BlockSpec((1, 16), lambda i: (_z, _z)),
    ],
    out_specs=pl.BlockSpec((BT, 16), lambda i: (i, _z)),
    out_shape=jax.ShapeDtypeStruct((N, 16), jnp.float32),
)


def kernel(x, edge_index, edge_type, w1, root1, b1, w2, root2, b2):
  src = edge_index[0].astype(jnp.int32)
  dst = edge_index[1].astype(jnp.int32)
  rel = edge_type.astype(jnp.int32)
  skey = dst * R + rel
  g2 = src * R + rel
  z2 = jnp.zeros((WB, 16), jnp.bfloat16)
  z1 = jnp.zeros((WB,), jnp.float32)
  S1, cnt = _sc_l1(skey, src, x.astype(jnp.bfloat16), z2, z1)
  cnt2 = cnt.reshape(N, R)
  h, h2 = _tc1(S1.reshape(N, 128), cnt2, x, w1, root1, b1.reshape(1, 32), w2)
  T2 = _sc_l2(skey, g2, h2.reshape(N * R, 16), z2)
  if isinstance(T2, (list, tuple)):
    T2 = T2[0]
  out = _tc2(T2.reshape(N, 128), cnt2, h, root2, b2.reshape(1, 16))
  return out
